# baseline (speedup 1.0000x reference)
import dataclasses
import functools

import jax
import jax.numpy as jnp
from jax import lax
from jax.experimental import pallas as pl
from jax.experimental.pallas import tpu as pltpu
from jax.experimental.pallas import tpu_sc as plsc

N_NODES = 10000
N_EDGES = 320000
D_FEAT = 128
HIDDEN = 64
D_GLOBAL = 16
NUM_GRAPHS = 64

NC = 2
NS = 16
NW = NC * NS
EPW = N_EDGES // NW
CHUNK = 64
NCHUNKS = -(-EPW // CHUNK)
EPW_PAD = NCHUNKS * CHUNK - EPW
Z_ROWS = N_NODES + NS
NBUF = 3
RPS = 624
RPS_LAST = N_NODES - RPS * (NS - 1)
DEG_CHUNK = 2000

BLK = 1024
GRID = (N_NODES + BLK - 1) // BLK


@functools.cache
def _sc_kernels():
    mesh = plsc.VectorSubcoreMesh(core_axis_name="c", subcore_axis_name="s")
    cp = pltpu.CompilerParams()
    if "needs_layout_passes" in pltpu.CompilerParams.__dataclass_fields__:
        cp = dataclasses.replace(cp, needs_layout_passes=False)
    cp_agg = pltpu.CompilerParams(use_tc_tiling_on_sc=False)

    @functools.partial(
        pl.kernel,
        out_type=jax.ShapeDtypeStruct((NW * N_NODES,), jnp.float32),
        mesh=mesh,
        compiler_params=cp,
        scratch_types=[
            pltpu.VMEM((N_NODES,), jnp.float32),
            pltpu.VMEM((DEG_CHUNK,), jnp.int32),
        ],
    )
    def sc_degree(dst_hbm, out_hbm, deg_v, idx_v):
        c = lax.axis_index("c")
        s = lax.axis_index("s")
        wid = s * NC + c

        @pl.loop(0, N_NODES, step=16)
        def _zero(i):
            deg_v[pl.ds(i, 16)] = jnp.zeros((16,), jnp.float32)

        ones = jnp.full((16,), 1.0, jnp.float32)
        base = wid * EPW

        @pl.loop(0, EPW, step=DEG_CHUNK)
        def _outer(e0):
            pltpu.sync_copy(dst_hbm.at[pl.ds(base + e0, DEG_CHUNK)], idx_v)

            @pl.loop(0, DEG_CHUNK, step=16)
            def _inner(j):
                plsc.addupdate_scatter(deg_v, [idx_v[pl.ds(j, 16)]], ones)

        pltpu.sync_copy(deg_v, out_hbm.at[pl.ds(wid * N_NODES, N_NODES)])

    @functools.partial(
        pl.kernel,
        out_type=jax.ShapeDtypeStruct((NC, N_NODES, HIDDEN), jnp.float32),
        mesh=mesh,
        compiler_params=cp_agg,
        scratch_types=[
            pltpu.VMEM_SHARED((Z_ROWS, HIDDEN), jnp.float32),
            pltpu.VMEM((NCHUNKS, CHUNK), jnp.int32),
            pltpu.VMEM((NCHUNKS, CHUNK), jnp.int32),
        ] + [pltpu.VMEM((CHUNK, HIDDEN), jnp.float32)] * NBUF
          + [pltpu.SemaphoreType.DMA] * NBUF,
    )
    def sc_edge_agg(src_hbm, dst_hbm, y_hbm, zero_hbm, out_hbm,
                    z_sh, src_v, dst_v, *bufs_and_sems):
        rbufs = bufs_and_sems[:NBUF]
        sems = bufs_and_sems[NBUF:]
        c = lax.axis_index("c")
        s = lax.axis_index("s")
        wid = s * NC + c

        pltpu.sync_copy(src_hbm.at[wid], src_v)
        pltpu.sync_copy(dst_hbm.at[wid], dst_v)

        for b in range(NBUF):
            pltpu.async_copy(y_hbm.at[src_v.at[b]], rbufs[b], sems[b])

        off = pl.multiple_of(s * RPS, 8)

        @pl.when(s < NS - 1)
        def _zero_main():
            pltpu.sync_copy(zero_hbm.at[pl.ds(off, RPS)],
                            z_sh.at[pl.ds(off, RPS)])

        @pl.when(s == NS - 1)
        def _zero_last():
            pltpu.sync_copy(zero_hbm.at[pl.ds(RPS * (NS - 1), RPS_LAST)],
                            z_sh.at[pl.ds(RPS * (NS - 1), RPS_LAST)])

        plsc.subcore_barrier()

        @pl.loop(0, (NCHUNKS // NBUF) * NBUF, step=NBUF)
        def _pipe(c0):
            for b in range(NBUF):
                pltpu.make_async_copy(
                    y_hbm.at[src_v.at[c0 + b]], rbufs[b], sems[b]).wait()
                pltpu.sync_copy(rbufs[b], z_sh.at[dst_v.at[c0 + b]], add=True)

                @pl.when(c0 + b + NBUF < NCHUNKS)
                def _refill():
                    pltpu.async_copy(
                        y_hbm.at[src_v.at[c0 + b + NBUF]], rbufs[b], sems[b])

        for cr in range((NCHUNKS // NBUF) * NBUF, NCHUNKS):
            b = cr % NBUF
            pltpu.make_async_copy(
                y_hbm.at[src_v.at[cr]], rbufs[b], sems[b]).wait()
            pltpu.sync_copy(rbufs[b], z_sh.at[dst_v.at[cr]], add=True)

        plsc.subcore_barrier()

        @pl.when(s < NS - 1)
        def _out_main():
            pltpu.sync_copy(z_sh.at[pl.ds(off, RPS)],
                            out_hbm.at[c, pl.ds(off, RPS)])

        @pl.when(s == NS - 1)
        def _out_last():
            pltpu.sync_copy(z_sh.at[pl.ds(RPS * (NS - 1), RPS_LAST)],
                            out_hbm.at[c, pl.ds(RPS * (NS - 1), RPS_LAST)])

    return sc_degree, sc_edge_agg


def _dis_from_parts(deg_ref):
    deg = jnp.sum(deg_ref[...], axis=0) + 1.0
    return lax.rsqrt(deg)


def _t1_body(x_ref, w1_ref, deg_ref, y_ref):
    xw = jnp.dot(x_ref[...], w1_ref[...], preferred_element_type=jnp.float32)
    dis = _dis_from_parts(deg_ref)
    y_ref[...] = xw * dis[:, None]


def _t2_body(z_ref, y1_ref, deg_ref, w2_ref, b1_ref, y2_ref):
    dis = _dis_from_parts(deg_ref)
    agg = (z_ref[0] + z_ref[1] + y1_ref[...]) * dis[:, None] + b1_ref[...]
    h1 = jnp.maximum(agg, 0.0)
    xw2 = jnp.dot(h1, w2_ref[...], preferred_element_type=jnp.float32)
    y2_ref[...] = xw2 * dis[:, None]


def _t3_body(z_ref, y2_ref, deg_ref, batch_ref, gf_ref, b2_ref,
             wg1_ref, bg1_ref, wg2_ref, bg2_ref,
             wc1_ref, bc1_ref, wc2_ref, bc2_ref,
             out_ref, pooled_acc, cnt_acc):
    i = pl.program_id(0)

    @pl.when(i == 0)
    def _init():
        pooled_acc[...] = jnp.zeros((NUM_GRAPHS, HIDDEN), jnp.float32)
        cnt_acc[...] = jnp.zeros((NUM_GRAPHS, 1), jnp.float32)

    dis = _dis_from_parts(deg_ref)
    h2 = (z_ref[0] + z_ref[1] + y2_ref[...]) * dis[:, None] + b2_ref[...]

    rowid = lax.broadcasted_iota(jnp.int32, (BLK, 1), 0) + i * BLK
    h2 = jnp.where(rowid < N_NODES, h2, 0.0)

    colid = lax.broadcasted_iota(jnp.int32, (1, BLK), 1) + i * BLK
    gids = lax.broadcasted_iota(jnp.int32, (NUM_GRAPHS, 1), 0)
    seg = jnp.where((batch_ref[...] == gids) & (colid < N_NODES), 1.0, 0.0)

    pooled_acc[...] += jnp.dot(seg, h2, preferred_element_type=jnp.float32)
    cnt_acc[...] += jnp.sum(seg, axis=1, keepdims=True)

    @pl.when(i == GRID - 1)
    def _finish():
        pooled = pooled_acc[...] / jnp.maximum(cnt_acc[...], 1.0)
        gh = jnp.maximum(
            jnp.dot(gf_ref[...], wg1_ref[...],
                    preferred_element_type=jnp.float32) + bg1_ref[...], 0.0)
        g = jnp.dot(gh, wg2_ref[...],
                    preferred_element_type=jnp.float32) + bg2_ref[...]
        combined = jnp.concatenate([pooled, g], axis=1)
        hc = jnp.maximum(
            jnp.dot(combined, wc1_ref[...],
                    preferred_element_type=jnp.float32) + bc1_ref[...], 0.0)
        out_ref[...] = jnp.dot(hc, wc2_ref[...],
                               preferred_element_type=jnp.float32) + bc2_ref[...]


def _full(shape):
    return pl.BlockSpec(shape, lambda i: tuple(0 for _ in shape))


def _tc_layer1(x, W1, deg_parts):
    return pl.pallas_call(
        _t1_body,
        grid=(GRID,),
        in_specs=[
            pl.BlockSpec((BLK, D_FEAT), lambda i: (i, 0)),
            _full((D_FEAT, HIDDEN)),
            pl.BlockSpec((NW, BLK), lambda i: (0, i)),
        ],
        out_specs=pl.BlockSpec((BLK, HIDDEN), lambda i: (i, 0)),
        out_shape=jax.ShapeDtypeStruct((N_NODES, HIDDEN), jnp.float32),
    )(x, W1, deg_parts)


def _tc_layer2(z1, y1, deg_parts, W2, b1):
    return pl.pallas_call(
        _t2_body,
        grid=(GRID,),
        in_specs=[
            pl.BlockSpec((NC, BLK, HIDDEN), lambda i: (0, i, 0)),
            pl.BlockSpec((BLK, HIDDEN), lambda i: (i, 0)),
            pl.BlockSpec((NW, BLK), lambda i: (0, i)),
            _full((HIDDEN, HIDDEN)),
            _full((1, HIDDEN)),
        ],
        out_specs=pl.BlockSpec((BLK, HIDDEN), lambda i: (i, 0)),
        out_shape=jax.ShapeDtypeStruct((N_NODES, HIDDEN), jnp.float32),
    )(z1, y1, deg_parts, W2, b1)


def _tc_final(z2, y2, deg_parts, batch2d, gf, b2,
              Wg1, bg1, Wg2, bg2, Wc1, bc1, Wc2, bc2):
    return pl.pallas_call(
        _t3_body,
        grid=(GRID,),
        in_specs=[
            pl.BlockSpec((NC, BLK, HIDDEN), lambda i: (0, i, 0)),
            pl.BlockSpec((BLK, HIDDEN), lambda i: (i, 0)),
            pl.BlockSpec((NW, BLK), lambda i: (0, i)),
            pl.BlockSpec((1, BLK), lambda i: (0, i)),
            _full((NUM_GRAPHS, D_GLOBAL)),
            _full((1, HIDDEN)),
            _full((D_GLOBAL, HIDDEN)),
            _full((1, HIDDEN)),
            _full((HIDDEN, HIDDEN)),
            _full((1, HIDDEN)),
            _full((2 * HIDDEN, HIDDEN)),
            _full((1, HIDDEN)),
            _full((HIDDEN, 1)),
            _full((1, 1)),
        ],
        out_specs=_full((NUM_GRAPHS, 1)),
        out_shape=jax.ShapeDtypeStruct((NUM_GRAPHS, 1), jnp.float32),
        scratch_shapes=[
            pltpu.VMEM((NUM_GRAPHS, HIDDEN), jnp.float32),
            pltpu.VMEM((NUM_GRAPHS, 1), jnp.float32),
        ],
    )(z2, y2, deg_parts, batch2d, gf, b2,
      Wg1, bg1, Wg2, bg2, Wc1, bc1, Wc2, bc2)


def kernel(x, edge_index, global_features, batch,
           W1, b1, W2, b2, Wg1, bg1, Wg2, bg2, Wc1, bc1, Wc2, bc2):
    ei = edge_index.astype(jnp.int32)
    src = ei[0]
    dst = ei[1]
    src3 = jnp.concatenate(
        [src.reshape(NW, EPW), jnp.zeros((NW, EPW_PAD), jnp.int32)],
        axis=1).reshape(NW, NCHUNKS, CHUNK)
    pad_rows = (N_NODES + jnp.arange(NW, dtype=jnp.int32) // NC)[:, None]
    dst3 = jnp.concatenate(
        [dst.reshape(NW, EPW),
         jnp.broadcast_to(pad_rows, (NW, EPW_PAD))],
        axis=1).reshape(NW, NCHUNKS, CHUNK)
    batch2d = batch.astype(jnp.int32).reshape(1, N_NODES)
    zeros = jnp.zeros((N_NODES, HIDDEN), jnp.float32)
    b1r = b1.reshape(1, HIDDEN)
    b2r = b2.reshape(1, HIDDEN)
    bg1r = bg1.reshape(1, HIDDEN)
    bg2r = bg2.reshape(1, HIDDEN)
    bc1r = bc1.reshape(1, HIDDEN)
    bc2r = bc2.reshape(1, 1)

    sc_degree, sc_edge_agg = _sc_kernels()
    deg_parts = sc_degree(dst).reshape(NW, N_NODES)
    y1 = _tc_layer1(x, W1, deg_parts)
    z1 = sc_edge_agg(src3, dst3, y1, zeros)
    y2 = _tc_layer2(z1, y1, deg_parts, W2, b1r)
    z2 = sc_edge_agg(src3, dst3, y2, zeros)
    return _tc_final(z2, y2, deg_parts, batch2d, global_features, b2r,
                     Wg1, bg1r, Wg2, bg2r, Wc1, bc1r, Wc2, bc2r)


# speedup vs baseline: 1.3909x; 1.3909x over previous
import dataclasses
import functools

import jax
import jax.numpy as jnp
from jax import lax
from jax.experimental import pallas as pl
from jax.experimental.pallas import tpu as pltpu
from jax.experimental.pallas import tpu_sc as plsc

N_NODES = 10000
N_EDGES = 320000
D_FEAT = 128
HIDDEN = 64
D_GLOBAL = 16
NUM_GRAPHS = 64

NC = 2
NS = 16
NW = NC * NS
EPW = N_EDGES // NW
CHUNK = 80
NCHUNKS = -(-EPW // CHUNK)
EPW_PAD = NCHUNKS * CHUNK - EPW
Z_ROWS = N_NODES + NS
NBUF = 4
RPS = 624
RPS_LAST = N_NODES - RPS * (NS - 1)
DEG_CHUNK = 2000

BLK = 1024
GRID = (N_NODES + BLK - 1) // BLK


@functools.cache
def _sc_kernels():
    mesh = plsc.VectorSubcoreMesh(core_axis_name="c", subcore_axis_name="s")
    cp = pltpu.CompilerParams()
    if "needs_layout_passes" in pltpu.CompilerParams.__dataclass_fields__:
        cp = dataclasses.replace(cp, needs_layout_passes=False)
    cp_agg = pltpu.CompilerParams(use_tc_tiling_on_sc=False)

    @functools.partial(
        pl.kernel,
        out_type=jax.ShapeDtypeStruct((NW * N_NODES,), jnp.float32),
        mesh=mesh,
        compiler_params=cp,
        scratch_types=[
            pltpu.VMEM((N_NODES,), jnp.float32),
            pltpu.VMEM((DEG_CHUNK,), jnp.int32),
        ],
    )
    def sc_degree(dst_hbm, out_hbm, deg_v, idx_v):
        c = lax.axis_index("c")
        s = lax.axis_index("s")
        wid = s * NC + c

        @pl.loop(0, N_NODES, step=16)
        def _zero(i):
            deg_v[pl.ds(i, 16)] = jnp.zeros((16,), jnp.float32)

        ones = jnp.full((16,), 1.0, jnp.float32)
        base = wid * EPW

        @pl.loop(0, EPW, step=DEG_CHUNK)
        def _outer(e0):
            pltpu.sync_copy(dst_hbm.at[pl.ds(base + e0, DEG_CHUNK)], idx_v)

            @pl.loop(0, DEG_CHUNK, step=16)
            def _inner(j):
                plsc.addupdate_scatter(deg_v, [idx_v[pl.ds(j, 16)]], ones)

        pltpu.sync_copy(deg_v, out_hbm.at[pl.ds(wid * N_NODES, N_NODES)])

    @functools.partial(
        pl.kernel,
        out_type=jax.ShapeDtypeStruct((NC, N_NODES, HIDDEN), jnp.float32),
        mesh=mesh,
        compiler_params=cp_agg,
        scratch_types=[
            pltpu.VMEM_SHARED((Z_ROWS, HIDDEN), jnp.float32),
            pltpu.VMEM((NCHUNKS, CHUNK), jnp.int32),
            pltpu.VMEM((NCHUNKS, CHUNK), jnp.int32),
        ] + [pltpu.VMEM((CHUNK, HIDDEN), jnp.float32)] * NBUF
          + [pltpu.SemaphoreType.DMA] * NBUF,
    )
    def sc_edge_agg(src_hbm, dst_hbm, y_hbm, zero_hbm, out_hbm,
                    z_sh, src_v, dst_v, *bufs_and_sems):
        rbufs = bufs_and_sems[:NBUF]
        sems = bufs_and_sems[NBUF:]
        c = lax.axis_index("c")
        s = lax.axis_index("s")
        wid = s * NC + c

        pltpu.sync_copy(src_hbm.at[wid], src_v)
        pltpu.sync_copy(dst_hbm.at[wid], dst_v)

        for b in range(NBUF):
            pltpu.async_copy(y_hbm.at[src_v.at[b]], rbufs[b], sems[b])

        off = pl.multiple_of(s * RPS, 8)

        @pl.when(s < NS - 1)
        def _zero_main():
            pltpu.sync_copy(zero_hbm.at[pl.ds(off, RPS)],
                            z_sh.at[pl.ds(off, RPS)])

        @pl.when(s == NS - 1)
        def _zero_last():
            pltpu.sync_copy(zero_hbm.at[pl.ds(RPS * (NS - 1), RPS_LAST)],
                            z_sh.at[pl.ds(RPS * (NS - 1), RPS_LAST)])

        plsc.subcore_barrier()

        @pl.loop(0, (NCHUNKS // NBUF) * NBUF, step=NBUF)
        def _pipe(c0):
            for b in range(NBUF):
                pltpu.make_async_copy(
                    y_hbm.at[src_v.at[c0 + b]], rbufs[b], sems[b]).wait()
                pltpu.sync_copy(rbufs[b], z_sh.at[dst_v.at[c0 + b]], add=True)

                @pl.when(c0 + b + NBUF < NCHUNKS)
                def _refill():
                    pltpu.async_copy(
                        y_hbm.at[src_v.at[c0 + b + NBUF]], rbufs[b], sems[b])

        for cr in range((NCHUNKS // NBUF) * NBUF, NCHUNKS):
            b = cr % NBUF
            pltpu.make_async_copy(
                y_hbm.at[src_v.at[cr]], rbufs[b], sems[b]).wait()
            pltpu.sync_copy(rbufs[b], z_sh.at[dst_v.at[cr]], add=True)

        plsc.subcore_barrier()

        @pl.when(s < NS - 1)
        def _out_main():
            pltpu.sync_copy(z_sh.at[pl.ds(off, RPS)],
                            out_hbm.at[c, pl.ds(off, RPS)])

        @pl.when(s == NS - 1)
        def _out_last():
            pltpu.sync_copy(z_sh.at[pl.ds(RPS * (NS - 1), RPS_LAST)],
                            out_hbm.at[c, pl.ds(RPS * (NS - 1), RPS_LAST)])

    return sc_degree, sc_edge_agg


def _dis_from_parts(deg_ref):
    deg = jnp.sum(deg_ref[...], axis=0) + 1.0
    return lax.rsqrt(deg)


def _t1_body(x_ref, w1_ref, deg_ref, y_ref):
    xw = jnp.dot(x_ref[...], w1_ref[...], preferred_element_type=jnp.float32)
    dis = _dis_from_parts(deg_ref)
    y_ref[...] = xw * dis[:, None]


def _t2_body(z_ref, y1_ref, deg_ref, w2_ref, b1_ref, y2_ref):
    dis = _dis_from_parts(deg_ref)
    agg = (z_ref[0] + z_ref[1] + y1_ref[...]) * dis[:, None] + b1_ref[...]
    h1 = jnp.maximum(agg, 0.0)
    xw2 = jnp.dot(h1, w2_ref[...], preferred_element_type=jnp.float32)
    y2_ref[...] = xw2 * dis[:, None]


def _t3_body(z_ref, y2_ref, deg_ref, batch_ref, gf_ref, b2_ref,
             wg1_ref, bg1_ref, wg2_ref, bg2_ref,
             wc1_ref, bc1_ref, wc2_ref, bc2_ref,
             out_ref, pooled_acc, cnt_acc):
    i = pl.program_id(0)

    @pl.when(i == 0)
    def _init():
        pooled_acc[...] = jnp.zeros((NUM_GRAPHS, HIDDEN), jnp.float32)
        cnt_acc[...] = jnp.zeros((NUM_GRAPHS, 1), jnp.float32)

    dis = _dis_from_parts(deg_ref)
    h2 = (z_ref[0] + z_ref[1] + y2_ref[...]) * dis[:, None] + b2_ref[...]

    rowid = lax.broadcasted_iota(jnp.int32, (BLK, 1), 0) + i * BLK
    h2 = jnp.where(rowid < N_NODES, h2, 0.0)

    colid = lax.broadcasted_iota(jnp.int32, (1, BLK), 1) + i * BLK
    gids = lax.broadcasted_iota(jnp.int32, (NUM_GRAPHS, 1), 0)
    seg = jnp.where((batch_ref[...] == gids) & (colid < N_NODES), 1.0, 0.0)

    pooled_acc[...] += jnp.dot(seg, h2, preferred_element_type=jnp.float32)
    cnt_acc[...] += jnp.sum(seg, axis=1, keepdims=True)

    @pl.when(i == GRID - 1)
    def _finish():
        pooled = pooled_acc[...] / jnp.maximum(cnt_acc[...], 1.0)
        gh = jnp.maximum(
            jnp.dot(gf_ref[...], wg1_ref[...],
                    preferred_element_type=jnp.float32) + bg1_ref[...], 0.0)
        g = jnp.dot(gh, wg2_ref[...],
                    preferred_element_type=jnp.float32) + bg2_ref[...]
        combined = jnp.concatenate([pooled, g], axis=1)
        hc = jnp.maximum(
            jnp.dot(combined, wc1_ref[...],
                    preferred_element_type=jnp.float32) + bc1_ref[...], 0.0)
        out_ref[...] = jnp.dot(hc, wc2_ref[...],
                               preferred_element_type=jnp.float32) + bc2_ref[...]


def _full(shape):
    return pl.BlockSpec(shape, lambda i: tuple(0 for _ in shape))


def _tc_layer1(x, W1, deg_parts):
    return pl.pallas_call(
        _t1_body,
        grid=(GRID,),
        in_specs=[
            pl.BlockSpec((BLK, D_FEAT), lambda i: (i, 0)),
            _full((D_FEAT, HIDDEN)),
            pl.BlockSpec((NW, BLK), lambda i: (0, i)),
        ],
        out_specs=pl.BlockSpec((BLK, HIDDEN), lambda i: (i, 0)),
        out_shape=jax.ShapeDtypeStruct((N_NODES, HIDDEN), jnp.float32),
    )(x, W1, deg_parts)


def _tc_layer2(z1, y1, deg_parts, W2, b1):
    return pl.pallas_call(
        _t2_body,
        grid=(GRID,),
        in_specs=[
            pl.BlockSpec((NC, BLK, HIDDEN), lambda i: (0, i, 0)),
            pl.BlockSpec((BLK, HIDDEN), lambda i: (i, 0)),
            pl.BlockSpec((NW, BLK), lambda i: (0, i)),
            _full((HIDDEN, HIDDEN)),
            _full((1, HIDDEN)),
        ],
        out_specs=pl.BlockSpec((BLK, HIDDEN), lambda i: (i, 0)),
        out_shape=jax.ShapeDtypeStruct((N_NODES, HIDDEN), jnp.float32),
    )(z1, y1, deg_parts, W2, b1)


def _tc_final(z2, y2, deg_parts, batch2d, gf, b2,
              Wg1, bg1, Wg2, bg2, Wc1, bc1, Wc2, bc2):
    return pl.pallas_call(
        _t3_body,
        grid=(GRID,),
        in_specs=[
            pl.BlockSpec((NC, BLK, HIDDEN), lambda i: (0, i, 0)),
            pl.BlockSpec((BLK, HIDDEN), lambda i: (i, 0)),
            pl.BlockSpec((NW, BLK), lambda i: (0, i)),
            pl.BlockSpec((1, BLK), lambda i: (0, i)),
            _full((NUM_GRAPHS, D_GLOBAL)),
            _full((1, HIDDEN)),
            _full((D_GLOBAL, HIDDEN)),
            _full((1, HIDDEN)),
            _full((HIDDEN, HIDDEN)),
            _full((1, HIDDEN)),
            _full((2 * HIDDEN, HIDDEN)),
            _full((1, HIDDEN)),
            _full((HIDDEN, 1)),
            _full((1, 1)),
        ],
        out_specs=_full((NUM_GRAPHS, 1)),
        out_shape=jax.ShapeDtypeStruct((NUM_GRAPHS, 1), jnp.float32),
        scratch_shapes=[
            pltpu.VMEM((NUM_GRAPHS, HIDDEN), jnp.float32),
            pltpu.VMEM((NUM_GRAPHS, 1), jnp.float32),
        ],
    )(z2, y2, deg_parts, batch2d, gf, b2,
      Wg1, bg1, Wg2, bg2, Wc1, bc1, Wc2, bc2)


def kernel(x, edge_index, global_features, batch,
           W1, b1, W2, b2, Wg1, bg1, Wg2, bg2, Wc1, bc1, Wc2, bc2):
    ei = edge_index.astype(jnp.int32)
    src = ei[0]
    dst = ei[1]
    src3 = jnp.concatenate(
        [src.reshape(NW, EPW), jnp.zeros((NW, EPW_PAD), jnp.int32)],
        axis=1).reshape(NW, NCHUNKS, CHUNK)
    pad_rows = (N_NODES + jnp.arange(NW, dtype=jnp.int32) // NC)[:, None]
    dst3 = jnp.concatenate(
        [dst.reshape(NW, EPW),
         jnp.broadcast_to(pad_rows, (NW, EPW_PAD))],
        axis=1).reshape(NW, NCHUNKS, CHUNK)
    batch2d = batch.astype(jnp.int32).reshape(1, N_NODES)
    zeros = jnp.zeros((N_NODES, HIDDEN), jnp.float32)
    b1r = b1.reshape(1, HIDDEN)
    b2r = b2.reshape(1, HIDDEN)
    bg1r = bg1.reshape(1, HIDDEN)
    bg2r = bg2.reshape(1, HIDDEN)
    bc1r = bc1.reshape(1, HIDDEN)
    bc2r = bc2.reshape(1, 1)

    sc_degree, sc_edge_agg = _sc_kernels()
    deg_parts = sc_degree(dst).reshape(NW, N_NODES)
    y1 = _tc_layer1(x, W1, deg_parts)
    z1 = sc_edge_agg(src3, dst3, y1, zeros)
    y2 = _tc_layer2(z1, y1, deg_parts, W2, b1r)
    z2 = sc_edge_agg(src3, dst3, y2, zeros)
    return _tc_final(z2, y2, deg_parts, batch2d, global_features, b2r,
                     Wg1, bg1r, Wg2, bg2r, Wc1, bc1r, Wc2, bc2r)


# speedup vs baseline: 1.4211x; 1.0217x over previous
import dataclasses
import functools

import jax
import jax.numpy as jnp
from jax import lax
from jax.experimental import pallas as pl
from jax.experimental.pallas import tpu as pltpu
from jax.experimental.pallas import tpu_sc as plsc

N_NODES = 10000
N_EDGES = 320000
D_FEAT = 128
HIDDEN = 64
D_GLOBAL = 16
NUM_GRAPHS = 64

NC = 2
NS = 16
NW = NC * NS
EPW = N_EDGES // NW
CHUNK = 80
NCHUNKS = -(-EPW // CHUNK)
EPW_PAD = NCHUNKS * CHUNK - EPW
Z_ROWS = N_NODES + NS
NBUF = 6
RPS = 624
RPS_LAST = N_NODES - RPS * (NS - 1)
DEG_CHUNK = 2000

BLK = 1024
GRID = (N_NODES + BLK - 1) // BLK


@functools.cache
def _sc_kernels():
    mesh = plsc.VectorSubcoreMesh(core_axis_name="c", subcore_axis_name="s")
    cp = pltpu.CompilerParams()
    if "needs_layout_passes" in pltpu.CompilerParams.__dataclass_fields__:
        cp = dataclasses.replace(cp, needs_layout_passes=False)
    cp_agg = pltpu.CompilerParams(use_tc_tiling_on_sc=False)

    @functools.partial(
        pl.kernel,
        out_type=jax.ShapeDtypeStruct((NW * N_NODES,), jnp.float32),
        mesh=mesh,
        compiler_params=cp,
        scratch_types=[
            pltpu.VMEM((N_NODES,), jnp.float32),
            pltpu.VMEM((DEG_CHUNK,), jnp.int32),
        ],
    )
    def sc_degree(dst_hbm, out_hbm, deg_v, idx_v):
        c = lax.axis_index("c")
        s = lax.axis_index("s")
        wid = s * NC + c

        @pl.loop(0, N_NODES, step=16)
        def _zero(i):
            deg_v[pl.ds(i, 16)] = jnp.zeros((16,), jnp.float32)

        ones = jnp.full((16,), 1.0, jnp.float32)
        base = wid * EPW

        @pl.loop(0, EPW, step=DEG_CHUNK)
        def _outer(e0):
            pltpu.sync_copy(dst_hbm.at[pl.ds(base + e0, DEG_CHUNK)], idx_v)

            @pl.loop(0, DEG_CHUNK, step=16)
            def _inner(j):
                plsc.addupdate_scatter(deg_v, [idx_v[pl.ds(j, 16)]], ones)

        pltpu.sync_copy(deg_v, out_hbm.at[pl.ds(wid * N_NODES, N_NODES)])

    @functools.partial(
        pl.kernel,
        out_type=jax.ShapeDtypeStruct((NC, N_NODES, HIDDEN), jnp.float32),
        mesh=mesh,
        compiler_params=cp_agg,
        scratch_types=[
            pltpu.VMEM_SHARED((Z_ROWS, HIDDEN), jnp.float32),
            pltpu.VMEM((NCHUNKS, CHUNK), jnp.int32),
            pltpu.VMEM((NCHUNKS, CHUNK), jnp.int32),
        ] + [pltpu.VMEM((CHUNK, HIDDEN), jnp.float32)] * NBUF
          + [pltpu.SemaphoreType.DMA] * NBUF,
    )
    def sc_edge_agg(src_hbm, dst_hbm, y_hbm, zero_hbm, out_hbm,
                    z_sh, src_v, dst_v, *bufs_and_sems):
        rbufs = bufs_and_sems[:NBUF]
        sems = bufs_and_sems[NBUF:]
        c = lax.axis_index("c")
        s = lax.axis_index("s")
        wid = s * NC + c

        pltpu.sync_copy(src_hbm.at[wid], src_v)
        pltpu.sync_copy(dst_hbm.at[wid], dst_v)

        for b in range(NBUF):
            pltpu.async_copy(y_hbm.at[src_v.at[b]], rbufs[b], sems[b])

        off = pl.multiple_of(s * RPS, 8)

        @pl.when(s < NS - 1)
        def _zero_main():
            pltpu.sync_copy(zero_hbm.at[pl.ds(off, RPS)],
                            z_sh.at[pl.ds(off, RPS)])

        @pl.when(s == NS - 1)
        def _zero_last():
            pltpu.sync_copy(zero_hbm.at[pl.ds(RPS * (NS - 1), RPS_LAST)],
                            z_sh.at[pl.ds(RPS * (NS - 1), RPS_LAST)])

        plsc.subcore_barrier()

        @pl.loop(0, (NCHUNKS // NBUF) * NBUF, step=NBUF)
        def _pipe(c0):
            for b in range(NBUF):
                pltpu.make_async_copy(
                    y_hbm.at[src_v.at[c0 + b]], rbufs[b], sems[b]).wait()
                pltpu.sync_copy(rbufs[b], z_sh.at[dst_v.at[c0 + b]], add=True)

                @pl.when(c0 + b + NBUF < NCHUNKS)
                def _refill():
                    pltpu.async_copy(
                        y_hbm.at[src_v.at[c0 + b + NBUF]], rbufs[b], sems[b])

        for cr in range((NCHUNKS // NBUF) * NBUF, NCHUNKS):
            b = cr % NBUF
            pltpu.make_async_copy(
                y_hbm.at[src_v.at[cr]], rbufs[b], sems[b]).wait()
            pltpu.sync_copy(rbufs[b], z_sh.at[dst_v.at[cr]], add=True)

        plsc.subcore_barrier()

        @pl.when(s < NS - 1)
        def _out_main():
            pltpu.sync_copy(z_sh.at[pl.ds(off, RPS)],
                            out_hbm.at[c, pl.ds(off, RPS)])

        @pl.when(s == NS - 1)
        def _out_last():
            pltpu.sync_copy(z_sh.at[pl.ds(RPS * (NS - 1), RPS_LAST)],
                            out_hbm.at[c, pl.ds(RPS * (NS - 1), RPS_LAST)])

    return sc_degree, sc_edge_agg


def _dis_from_parts(deg_ref):
    deg = jnp.sum(deg_ref[...], axis=0) + 1.0
    return lax.rsqrt(deg)


def _t1_body(x_ref, w1_ref, deg_ref, y_ref):
    xw = jnp.dot(x_ref[...], w1_ref[...], preferred_element_type=jnp.float32)
    dis = _dis_from_parts(deg_ref)
    y_ref[...] = xw * dis[:, None]


def _t2_body(z_ref, y1_ref, deg_ref, w2_ref, b1_ref, y2_ref):
    dis = _dis_from_parts(deg_ref)
    agg = (z_ref[0] + z_ref[1] + y1_ref[...]) * dis[:, None] + b1_ref[...]
    h1 = jnp.maximum(agg, 0.0)
    xw2 = jnp.dot(h1, w2_ref[...], preferred_element_type=jnp.float32)
    y2_ref[...] = xw2 * dis[:, None]


def _t3_body(z_ref, y2_ref, deg_ref, batch_ref, gf_ref, b2_ref,
             wg1_ref, bg1_ref, wg2_ref, bg2_ref,
             wc1_ref, bc1_ref, wc2_ref, bc2_ref,
             out_ref, pooled_acc, cnt_acc):
    i = pl.program_id(0)

    @pl.when(i == 0)
    def _init():
        pooled_acc[...] = jnp.zeros((NUM_GRAPHS, HIDDEN), jnp.float32)
        cnt_acc[...] = jnp.zeros((NUM_GRAPHS, 1), jnp.float32)

    dis = _dis_from_parts(deg_ref)
    h2 = (z_ref[0] + z_ref[1] + y2_ref[...]) * dis[:, None] + b2_ref[...]

    rowid = lax.broadcasted_iota(jnp.int32, (BLK, 1), 0) + i * BLK
    h2 = jnp.where(rowid < N_NODES, h2, 0.0)

    colid = lax.broadcasted_iota(jnp.int32, (1, BLK), 1) + i * BLK
    gids = lax.broadcasted_iota(jnp.int32, (NUM_GRAPHS, 1), 0)
    seg = jnp.where((batch_ref[...] == gids) & (colid < N_NODES), 1.0, 0.0)

    pooled_acc[...] += jnp.dot(seg, h2, preferred_element_type=jnp.float32)
    cnt_acc[...] += jnp.sum(seg, axis=1, keepdims=True)

    @pl.when(i == GRID - 1)
    def _finish():
        pooled = pooled_acc[...] / jnp.maximum(cnt_acc[...], 1.0)
        gh = jnp.maximum(
            jnp.dot(gf_ref[...], wg1_ref[...],
                    preferred_element_type=jnp.float32) + bg1_ref[...], 0.0)
        g = jnp.dot(gh, wg2_ref[...],
                    preferred_element_type=jnp.float32) + bg2_ref[...]
        combined = jnp.concatenate([pooled, g], axis=1)
        hc = jnp.maximum(
            jnp.dot(combined, wc1_ref[...],
                    preferred_element_type=jnp.float32) + bc1_ref[...], 0.0)
        out_ref[...] = jnp.dot(hc, wc2_ref[...],
                               preferred_element_type=jnp.float32) + bc2_ref[...]


def _full(shape):
    return pl.BlockSpec(shape, lambda i: tuple(0 for _ in shape))


def _tc_layer1(x, W1, deg_parts):
    return pl.pallas_call(
        _t1_body,
        grid=(GRID,),
        in_specs=[
            pl.BlockSpec((BLK, D_FEAT), lambda i: (i, 0)),
            _full((D_FEAT, HIDDEN)),
            pl.BlockSpec((NW, BLK), lambda i: (0, i)),
        ],
        out_specs=pl.BlockSpec((BLK, HIDDEN), lambda i: (i, 0)),
        out_shape=jax.ShapeDtypeStruct((N_NODES, HIDDEN), jnp.float32),
    )(x, W1, deg_parts)


def _tc_layer2(z1, y1, deg_parts, W2, b1):
    return pl.pallas_call(
        _t2_body,
        grid=(GRID,),
        in_specs=[
            pl.BlockSpec((NC, BLK, HIDDEN), lambda i: (0, i, 0)),
            pl.BlockSpec((BLK, HIDDEN), lambda i: (i, 0)),
            pl.BlockSpec((NW, BLK), lambda i: (0, i)),
            _full((HIDDEN, HIDDEN)),
            _full((1, HIDDEN)),
        ],
        out_specs=pl.BlockSpec((BLK, HIDDEN), lambda i: (i, 0)),
        out_shape=jax.ShapeDtypeStruct((N_NODES, HIDDEN), jnp.float32),
    )(z1, y1, deg_parts, W2, b1)


def _tc_final(z2, y2, deg_parts, batch2d, gf, b2,
              Wg1, bg1, Wg2, bg2, Wc1, bc1, Wc2, bc2):
    return pl.pallas_call(
        _t3_body,
        grid=(GRID,),
        in_specs=[
            pl.BlockSpec((NC, BLK, HIDDEN), lambda i: (0, i, 0)),
            pl.BlockSpec((BLK, HIDDEN), lambda i: (i, 0)),
            pl.BlockSpec((NW, BLK), lambda i: (0, i)),
            pl.BlockSpec((1, BLK), lambda i: (0, i)),
            _full((NUM_GRAPHS, D_GLOBAL)),
            _full((1, HIDDEN)),
            _full((D_GLOBAL, HIDDEN)),
            _full((1, HIDDEN)),
            _full((HIDDEN, HIDDEN)),
            _full((1, HIDDEN)),
            _full((2 * HIDDEN, HIDDEN)),
            _full((1, HIDDEN)),
            _full((HIDDEN, 1)),
            _full((1, 1)),
        ],
        out_specs=_full((NUM_GRAPHS, 1)),
        out_shape=jax.ShapeDtypeStruct((NUM_GRAPHS, 1), jnp.float32),
        scratch_shapes=[
            pltpu.VMEM((NUM_GRAPHS, HIDDEN), jnp.float32),
            pltpu.VMEM((NUM_GRAPHS, 1), jnp.float32),
        ],
    )(z2, y2, deg_parts, batch2d, gf, b2,
      Wg1, bg1, Wg2, bg2, Wc1, bc1, Wc2, bc2)


def kernel(x, edge_index, global_features, batch,
           W1, b1, W2, b2, Wg1, bg1, Wg2, bg2, Wc1, bc1, Wc2, bc2):
    ei = edge_index.astype(jnp.int32)
    src = ei[0]
    dst = ei[1]
    src3 = jnp.concatenate(
        [src.reshape(NW, EPW), jnp.zeros((NW, EPW_PAD), jnp.int32)],
        axis=1).reshape(NW, NCHUNKS, CHUNK)
    pad_rows = (N_NODES + jnp.arange(NW, dtype=jnp.int32) // NC)[:, None]
    dst3 = jnp.concatenate(
        [dst.reshape(NW, EPW),
         jnp.broadcast_to(pad_rows, (NW, EPW_PAD))],
        axis=1).reshape(NW, NCHUNKS, CHUNK)
    batch2d = batch.astype(jnp.int32).reshape(1, N_NODES)
    zeros = jnp.zeros((N_NODES, HIDDEN), jnp.float32)
    b1r = b1.reshape(1, HIDDEN)
    b2r = b2.reshape(1, HIDDEN)
    bg1r = bg1.reshape(1, HIDDEN)
    bg2r = bg2.reshape(1, HIDDEN)
    bc1r = bc1.reshape(1, HIDDEN)
    bc2r = bc2.reshape(1, 1)

    sc_degree, sc_edge_agg = _sc_kernels()
    deg_parts = sc_degree(dst).reshape(NW, N_NODES)
    y1 = _tc_layer1(x, W1, deg_parts)
    z1 = sc_edge_agg(src3, dst3, y1, zeros)
    y2 = _tc_layer2(z1, y1, deg_parts, W2, b1r)
    z2 = sc_edge_agg(src3, dst3, y2, zeros)
    return _tc_final(z2, y2, deg_parts, batch2d, global_features, b2r,
                     Wg1, bg1r, Wg2, bg2r, Wc1, bc1r, Wc2, bc2r)


# speedup vs baseline: 1.7064x; 1.2008x over previous
import dataclasses
import functools

import jax
import jax.numpy as jnp
from jax import lax
from jax.experimental import pallas as pl
from jax.experimental.pallas import tpu as pltpu
from jax.experimental.pallas import tpu_sc as plsc

N_NODES = 10000
N_EDGES = 320000
D_FEAT = 128
HIDDEN = 64
D_GLOBAL = 16
NUM_GRAPHS = 64

NC = 2
NS = 16
NW = NC * NS
EPW = N_EDGES // NW
CHUNK = 80
NCHUNKS = -(-EPW // CHUNK)
EPW_PAD = NCHUNKS * CHUNK - EPW
Z_ROWS = N_NODES + NS
NBUF = 6
EDGE_DT = jnp.bfloat16
RPS = 624
RPS_LAST = N_NODES - RPS * (NS - 1)
DEG_CHUNK = 2000

BLK = 1024
GRID = (N_NODES + BLK - 1) // BLK


@functools.cache
def _sc_kernels():
    mesh = plsc.VectorSubcoreMesh(core_axis_name="c", subcore_axis_name="s")
    cp = pltpu.CompilerParams()
    if "needs_layout_passes" in pltpu.CompilerParams.__dataclass_fields__:
        cp = dataclasses.replace(cp, needs_layout_passes=False)
    cp_agg = pltpu.CompilerParams(use_tc_tiling_on_sc=False)

    @functools.partial(
        pl.kernel,
        out_type=jax.ShapeDtypeStruct((NW * N_NODES,), jnp.float32),
        mesh=mesh,
        compiler_params=cp,
        scratch_types=[
            pltpu.VMEM((N_NODES,), jnp.float32),
            pltpu.VMEM((DEG_CHUNK,), jnp.int32),
        ],
    )
    def sc_degree(dst_hbm, out_hbm, deg_v, idx_v):
        c = lax.axis_index("c")
        s = lax.axis_index("s")
        wid = s * NC + c

        @pl.loop(0, N_NODES, step=16)
        def _zero(i):
            deg_v[pl.ds(i, 16)] = jnp.zeros((16,), jnp.float32)

        ones = jnp.full((16,), 1.0, jnp.float32)
        base = wid * EPW

        @pl.loop(0, EPW, step=DEG_CHUNK)
        def _outer(e0):
            pltpu.sync_copy(dst_hbm.at[pl.ds(base + e0, DEG_CHUNK)], idx_v)

            @pl.loop(0, DEG_CHUNK, step=16)
            def _inner(j):
                plsc.addupdate_scatter(deg_v, [idx_v[pl.ds(j, 16)]], ones)

        pltpu.sync_copy(deg_v, out_hbm.at[pl.ds(wid * N_NODES, N_NODES)])

    @functools.partial(
        pl.kernel,
        out_type=jax.ShapeDtypeStruct((NC, N_NODES, HIDDEN), EDGE_DT),
        mesh=mesh,
        compiler_params=cp_agg,
        scratch_types=[
            pltpu.VMEM_SHARED((Z_ROWS, HIDDEN), EDGE_DT),
            pltpu.VMEM((NCHUNKS, CHUNK), jnp.int32),
            pltpu.VMEM((NCHUNKS, CHUNK), jnp.int32),
        ] + [pltpu.VMEM((CHUNK, HIDDEN), EDGE_DT)] * NBUF
          + [pltpu.SemaphoreType.DMA] * NBUF,
    )
    def sc_edge_agg(src_hbm, dst_hbm, y_hbm, zero_hbm, out_hbm,
                    z_sh, src_v, dst_v, *bufs_and_sems):
        rbufs = bufs_and_sems[:NBUF]
        sems = bufs_and_sems[NBUF:]
        c = lax.axis_index("c")
        s = lax.axis_index("s")
        wid = s * NC + c

        pltpu.sync_copy(src_hbm.at[wid], src_v)
        pltpu.sync_copy(dst_hbm.at[wid], dst_v)

        for b in range(NBUF):
            pltpu.async_copy(y_hbm.at[src_v.at[b]], rbufs[b], sems[b])

        off = pl.multiple_of(s * RPS, 8)

        @pl.when(s < NS - 1)
        def _zero_main():
            pltpu.sync_copy(zero_hbm.at[pl.ds(off, RPS)],
                            z_sh.at[pl.ds(off, RPS)])

        @pl.when(s == NS - 1)
        def _zero_last():
            pltpu.sync_copy(zero_hbm.at[pl.ds(RPS * (NS - 1), RPS_LAST)],
                            z_sh.at[pl.ds(RPS * (NS - 1), RPS_LAST)])

        plsc.subcore_barrier()

        @pl.loop(0, (NCHUNKS // NBUF) * NBUF, step=NBUF)
        def _pipe(c0):
            for b in range(NBUF):
                pltpu.make_async_copy(
                    y_hbm.at[src_v.at[c0 + b]], rbufs[b], sems[b]).wait()
                pltpu.sync_copy(rbufs[b], z_sh.at[dst_v.at[c0 + b]], add=True)

                @pl.when(c0 + b + NBUF < NCHUNKS)
                def _refill():
                    pltpu.async_copy(
                        y_hbm.at[src_v.at[c0 + b + NBUF]], rbufs[b], sems[b])

        for cr in range((NCHUNKS // NBUF) * NBUF, NCHUNKS):
            b = cr % NBUF
            pltpu.make_async_copy(
                y_hbm.at[src_v.at[cr]], rbufs[b], sems[b]).wait()
            pltpu.sync_copy(rbufs[b], z_sh.at[dst_v.at[cr]], add=True)

        plsc.subcore_barrier()

        @pl.when(s < NS - 1)
        def _out_main():
            pltpu.sync_copy(z_sh.at[pl.ds(off, RPS)],
                            out_hbm.at[c, pl.ds(off, RPS)])

        @pl.when(s == NS - 1)
        def _out_last():
            pltpu.sync_copy(z_sh.at[pl.ds(RPS * (NS - 1), RPS_LAST)],
                            out_hbm.at[c, pl.ds(RPS * (NS - 1), RPS_LAST)])

    return sc_degree, sc_edge_agg


def _dis_from_parts(deg_ref):
    deg = jnp.sum(deg_ref[...], axis=0) + 1.0
    return lax.rsqrt(deg)


def _t1_body(x_ref, w1_ref, deg_ref, y_ref):
    xw = jnp.dot(x_ref[...], w1_ref[...], preferred_element_type=jnp.float32)
    dis = _dis_from_parts(deg_ref)
    y_ref[...] = (xw * dis[:, None]).astype(EDGE_DT)


def _t2_body(z_ref, y1_ref, deg_ref, w2_ref, b1_ref, y2_ref):
    dis = _dis_from_parts(deg_ref)
    z = (z_ref[0] + z_ref[1] + y1_ref[...]).astype(jnp.float32)
    agg = z * dis[:, None] + b1_ref[...]
    h1 = jnp.maximum(agg, 0.0)
    xw2 = jnp.dot(h1, w2_ref[...], preferred_element_type=jnp.float32)
    y2_ref[...] = (xw2 * dis[:, None]).astype(EDGE_DT)


def _t3_body(z_ref, y2_ref, deg_ref, batch_ref, gf_ref, b2_ref,
             wg1_ref, bg1_ref, wg2_ref, bg2_ref,
             wc1_ref, bc1_ref, wc2_ref, bc2_ref,
             out_ref, pooled_acc, cnt_acc):
    i = pl.program_id(0)

    @pl.when(i == 0)
    def _init():
        pooled_acc[...] = jnp.zeros((NUM_GRAPHS, HIDDEN), jnp.float32)
        cnt_acc[...] = jnp.zeros((NUM_GRAPHS, 1), jnp.float32)

    dis = _dis_from_parts(deg_ref)
    z = (z_ref[0] + z_ref[1] + y2_ref[...]).astype(jnp.float32)
    h2 = z * dis[:, None] + b2_ref[...]

    rowid = lax.broadcasted_iota(jnp.int32, (BLK, 1), 0) + i * BLK
    h2 = jnp.where(rowid < N_NODES, h2, 0.0)

    colid = lax.broadcasted_iota(jnp.int32, (1, BLK), 1) + i * BLK
    gids = lax.broadcasted_iota(jnp.int32, (NUM_GRAPHS, 1), 0)
    seg = jnp.where((batch_ref[...] == gids) & (colid < N_NODES), 1.0, 0.0)

    pooled_acc[...] += jnp.dot(seg, h2, preferred_element_type=jnp.float32)
    cnt_acc[...] += jnp.sum(seg, axis=1, keepdims=True)

    @pl.when(i == GRID - 1)
    def _finish():
        pooled = pooled_acc[...] / jnp.maximum(cnt_acc[...], 1.0)
        gh = jnp.maximum(
            jnp.dot(gf_ref[...], wg1_ref[...],
                    preferred_element_type=jnp.float32) + bg1_ref[...], 0.0)
        g = jnp.dot(gh, wg2_ref[...],
                    preferred_element_type=jnp.float32) + bg2_ref[...]
        combined = jnp.concatenate([pooled, g], axis=1)
        hc = jnp.maximum(
            jnp.dot(combined, wc1_ref[...],
                    preferred_element_type=jnp.float32) + bc1_ref[...], 0.0)
        out_ref[...] = jnp.dot(hc, wc2_ref[...],
                               preferred_element_type=jnp.float32) + bc2_ref[...]


def _full(shape):
    return pl.BlockSpec(shape, lambda i: tuple(0 for _ in shape))


def _tc_layer1(x, W1, deg_parts):
    return pl.pallas_call(
        _t1_body,
        grid=(GRID,),
        in_specs=[
            pl.BlockSpec((BLK, D_FEAT), lambda i: (i, 0)),
            _full((D_FEAT, HIDDEN)),
            pl.BlockSpec((NW, BLK), lambda i: (0, i)),
        ],
        out_specs=pl.BlockSpec((BLK, HIDDEN), lambda i: (i, 0)),
        out_shape=jax.ShapeDtypeStruct((N_NODES, HIDDEN), EDGE_DT),
    )(x, W1, deg_parts)


def _tc_layer2(z1, y1, deg_parts, W2, b1):
    return pl.pallas_call(
        _t2_body,
        grid=(GRID,),
        in_specs=[
            pl.BlockSpec((NC, BLK, HIDDEN), lambda i: (0, i, 0)),
            pl.BlockSpec((BLK, HIDDEN), lambda i: (i, 0)),
            pl.BlockSpec((NW, BLK), lambda i: (0, i)),
            _full((HIDDEN, HIDDEN)),
            _full((1, HIDDEN)),
        ],
        out_specs=pl.BlockSpec((BLK, HIDDEN), lambda i: (i, 0)),
        out_shape=jax.ShapeDtypeStruct((N_NODES, HIDDEN), EDGE_DT),
    )(z1, y1, deg_parts, W2, b1)


def _tc_final(z2, y2, deg_parts, batch2d, gf, b2,
              Wg1, bg1, Wg2, bg2, Wc1, bc1, Wc2, bc2):
    return pl.pallas_call(
        _t3_body,
        grid=(GRID,),
        in_specs=[
            pl.BlockSpec((NC, BLK, HIDDEN), lambda i: (0, i, 0)),
            pl.BlockSpec((BLK, HIDDEN), lambda i: (i, 0)),
            pl.BlockSpec((NW, BLK), lambda i: (0, i)),
            pl.BlockSpec((1, BLK), lambda i: (0, i)),
            _full((NUM_GRAPHS, D_GLOBAL)),
            _full((1, HIDDEN)),
            _full((D_GLOBAL, HIDDEN)),
            _full((1, HIDDEN)),
            _full((HIDDEN, HIDDEN)),
            _full((1, HIDDEN)),
            _full((2 * HIDDEN, HIDDEN)),
            _full((1, HIDDEN)),
            _full((HIDDEN, 1)),
            _full((1, 1)),
        ],
        out_specs=_full((NUM_GRAPHS, 1)),
        out_shape=jax.ShapeDtypeStruct((NUM_GRAPHS, 1), jnp.float32),
        scratch_shapes=[
            pltpu.VMEM((NUM_GRAPHS, HIDDEN), jnp.float32),
            pltpu.VMEM((NUM_GRAPHS, 1), jnp.float32),
        ],
    )(z2, y2, deg_parts, batch2d, gf, b2,
      Wg1, bg1, Wg2, bg2, Wc1, bc1, Wc2, bc2)


def kernel(x, edge_index, global_features, batch,
           W1, b1, W2, b2, Wg1, bg1, Wg2, bg2, Wc1, bc1, Wc2, bc2):
    ei = edge_index.astype(jnp.int32)
    src = ei[0]
    dst = ei[1]
    src3 = jnp.concatenate(
        [src.reshape(NW, EPW), jnp.zeros((NW, EPW_PAD), jnp.int32)],
        axis=1).reshape(NW, NCHUNKS, CHUNK)
    pad_rows = (N_NODES + jnp.arange(NW, dtype=jnp.int32) // NC)[:, None]
    dst3 = jnp.concatenate(
        [dst.reshape(NW, EPW),
         jnp.broadcast_to(pad_rows, (NW, EPW_PAD))],
        axis=1).reshape(NW, NCHUNKS, CHUNK)
    batch2d = batch.astype(jnp.int32).reshape(1, N_NODES)
    zeros = jnp.zeros((N_NODES, HIDDEN), EDGE_DT)
    b1r = b1.reshape(1, HIDDEN)
    b2r = b2.reshape(1, HIDDEN)
    bg1r = bg1.reshape(1, HIDDEN)
    bg2r = bg2.reshape(1, HIDDEN)
    bc1r = bc1.reshape(1, HIDDEN)
    bc2r = bc2.reshape(1, 1)

    sc_degree, sc_edge_agg = _sc_kernels()
    deg_parts = sc_degree(dst).reshape(NW, N_NODES)
    y1 = _tc_layer1(x, W1, deg_parts)
    z1 = sc_edge_agg(src3, dst3, y1, zeros)
    y2 = _tc_layer2(z1, y1, deg_parts, W2, b1r)
    z2 = sc_edge_agg(src3, dst3, y2, zeros)
    return _tc_final(z2, y2, deg_parts, batch2d, global_features, b2r,
                     Wg1, bg1r, Wg2, bg2r, Wc1, bc1r, Wc2, bc2r)


# speedup vs baseline: 1.7341x; 1.0162x over previous
import dataclasses
import functools

import jax
import jax.numpy as jnp
from jax import lax
from jax.experimental import pallas as pl
from jax.experimental.pallas import tpu as pltpu
from jax.experimental.pallas import tpu_sc as plsc

N_NODES = 10000
N_EDGES = 320000
D_FEAT = 128
HIDDEN = 64
D_GLOBAL = 16
NUM_GRAPHS = 64

NC = 2
NS = 16
NW = NC * NS
EPW = N_EDGES // NW
CHUNK = 80
NCHUNKS = -(-EPW // CHUNK)
EPW_PAD = NCHUNKS * CHUNK - EPW
Z_ROWS = N_NODES + NS
NBUF = 8
EDGE_DT = jnp.bfloat16
RPS = 624
RPS_LAST = N_NODES - RPS * (NS - 1)
DEG_CHUNK = 2000

BLK = 1024
GRID = (N_NODES + BLK - 1) // BLK


@functools.cache
def _sc_kernels():
    mesh = plsc.VectorSubcoreMesh(core_axis_name="c", subcore_axis_name="s")
    cp = pltpu.CompilerParams()
    if "needs_layout_passes" in pltpu.CompilerParams.__dataclass_fields__:
        cp = dataclasses.replace(cp, needs_layout_passes=False)
    cp_agg = pltpu.CompilerParams(use_tc_tiling_on_sc=False)

    @functools.partial(
        pl.kernel,
        out_type=jax.ShapeDtypeStruct((NW * N_NODES,), jnp.float32),
        mesh=mesh,
        compiler_params=cp,
        scratch_types=[
            pltpu.VMEM((N_NODES,), jnp.float32),
            pltpu.VMEM((DEG_CHUNK,), jnp.int32),
        ],
    )
    def sc_degree(dst_hbm, out_hbm, deg_v, idx_v):
        c = lax.axis_index("c")
        s = lax.axis_index("s")
        wid = s * NC + c

        @pl.loop(0, N_NODES, step=16)
        def _zero(i):
            deg_v[pl.ds(i, 16)] = jnp.zeros((16,), jnp.float32)

        ones = jnp.full((16,), 1.0, jnp.float32)
        base = wid * EPW

        @pl.loop(0, EPW, step=DEG_CHUNK)
        def _outer(e0):
            pltpu.sync_copy(dst_hbm.at[pl.ds(base + e0, DEG_CHUNK)], idx_v)

            @pl.loop(0, DEG_CHUNK, step=16)
            def _inner(j):
                plsc.addupdate_scatter(deg_v, [idx_v[pl.ds(j, 16)]], ones)

        pltpu.sync_copy(deg_v, out_hbm.at[pl.ds(wid * N_NODES, N_NODES)])

    @functools.partial(
        pl.kernel,
        out_type=jax.ShapeDtypeStruct((NC, N_NODES, HIDDEN), EDGE_DT),
        mesh=mesh,
        compiler_params=cp_agg,
        scratch_types=[
            pltpu.VMEM_SHARED((Z_ROWS, HIDDEN), EDGE_DT),
            pltpu.VMEM((NCHUNKS, CHUNK), jnp.int32),
            pltpu.VMEM((NCHUNKS, CHUNK), jnp.int32),
        ] + [pltpu.VMEM((CHUNK, HIDDEN), EDGE_DT)] * NBUF
          + [pltpu.SemaphoreType.DMA] * NBUF,
    )
    def sc_edge_agg(src_hbm, dst_hbm, y_hbm, zero_hbm, out_hbm,
                    z_sh, src_v, dst_v, *bufs_and_sems):
        rbufs = bufs_and_sems[:NBUF]
        sems = bufs_and_sems[NBUF:]
        c = lax.axis_index("c")
        s = lax.axis_index("s")
        wid = s * NC + c

        pltpu.sync_copy(src_hbm.at[wid], src_v)
        pltpu.sync_copy(dst_hbm.at[wid], dst_v)

        for b in range(NBUF):
            pltpu.async_copy(y_hbm.at[src_v.at[b]], rbufs[b], sems[b])

        off = pl.multiple_of(s * RPS, 8)

        @pl.when(s < NS - 1)
        def _zero_main():
            pltpu.sync_copy(zero_hbm.at[pl.ds(off, RPS)],
                            z_sh.at[pl.ds(off, RPS)])

        @pl.when(s == NS - 1)
        def _zero_last():
            pltpu.sync_copy(zero_hbm.at[pl.ds(RPS * (NS - 1), RPS_LAST)],
                            z_sh.at[pl.ds(RPS * (NS - 1), RPS_LAST)])

        plsc.subcore_barrier()

        @pl.loop(0, (NCHUNKS // NBUF) * NBUF, step=NBUF)
        def _pipe(c0):
            for b in range(NBUF):
                pltpu.make_async_copy(
                    y_hbm.at[src_v.at[c0 + b]], rbufs[b], sems[b]).wait()
                pltpu.sync_copy(rbufs[b], z_sh.at[dst_v.at[c0 + b]], add=True)

                @pl.when(c0 + b + NBUF < NCHUNKS)
                def _refill():
                    pltpu.async_copy(
                        y_hbm.at[src_v.at[c0 + b + NBUF]], rbufs[b], sems[b])

        for cr in range((NCHUNKS // NBUF) * NBUF, NCHUNKS):
            b = cr % NBUF
            pltpu.make_async_copy(
                y_hbm.at[src_v.at[cr]], rbufs[b], sems[b]).wait()
            pltpu.sync_copy(rbufs[b], z_sh.at[dst_v.at[cr]], add=True)

        plsc.subcore_barrier()

        @pl.when(s < NS - 1)
        def _out_main():
            pltpu.sync_copy(z_sh.at[pl.ds(off, RPS)],
                            out_hbm.at[c, pl.ds(off, RPS)])

        @pl.when(s == NS - 1)
        def _out_last():
            pltpu.sync_copy(z_sh.at[pl.ds(RPS * (NS - 1), RPS_LAST)],
                            out_hbm.at[c, pl.ds(RPS * (NS - 1), RPS_LAST)])

    return sc_degree, sc_edge_agg


def _dis_from_parts(deg_ref):
    deg = jnp.sum(deg_ref[...], axis=0) + 1.0
    return lax.rsqrt(deg)


def _t1_body(x_ref, w1_ref, deg_ref, y_ref):
    xw = jnp.dot(x_ref[...], w1_ref[...], preferred_element_type=jnp.float32)
    dis = _dis_from_parts(deg_ref)
    y_ref[...] = (xw * dis[:, None]).astype(EDGE_DT)


def _t2_body(z_ref, y1_ref, deg_ref, w2_ref, b1_ref, y2_ref):
    dis = _dis_from_parts(deg_ref)
    z = (z_ref[0] + z_ref[1] + y1_ref[...]).astype(jnp.float32)
    agg = z * dis[:, None] + b1_ref[...]
    h1 = jnp.maximum(agg, 0.0)
    xw2 = jnp.dot(h1, w2_ref[...], preferred_element_type=jnp.float32)
    y2_ref[...] = (xw2 * dis[:, None]).astype(EDGE_DT)


def _t3_body(z_ref, y2_ref, deg_ref, batch_ref, gf_ref, b2_ref,
             wg1_ref, bg1_ref, wg2_ref, bg2_ref,
             wc1_ref, bc1_ref, wc2_ref, bc2_ref,
             out_ref, pooled_acc, cnt_acc):
    i = pl.program_id(0)

    @pl.when(i == 0)
    def _init():
        pooled_acc[...] = jnp.zeros((NUM_GRAPHS, HIDDEN), jnp.float32)
        cnt_acc[...] = jnp.zeros((NUM_GRAPHS, 1), jnp.float32)

    dis = _dis_from_parts(deg_ref)
    z = (z_ref[0] + z_ref[1] + y2_ref[...]).astype(jnp.float32)
    h2 = z * dis[:, None] + b2_ref[...]

    rowid = lax.broadcasted_iota(jnp.int32, (BLK, 1), 0) + i * BLK
    h2 = jnp.where(rowid < N_NODES, h2, 0.0)

    colid = lax.broadcasted_iota(jnp.int32, (1, BLK), 1) + i * BLK
    gids = lax.broadcasted_iota(jnp.int32, (NUM_GRAPHS, 1), 0)
    seg = jnp.where((batch_ref[...] == gids) & (colid < N_NODES), 1.0, 0.0)

    pooled_acc[...] += jnp.dot(seg, h2, preferred_element_type=jnp.float32)
    cnt_acc[...] += jnp.sum(seg, axis=1, keepdims=True)

    @pl.when(i == GRID - 1)
    def _finish():
        pooled = pooled_acc[...] / jnp.maximum(cnt_acc[...], 1.0)
        gh = jnp.maximum(
            jnp.dot(gf_ref[...], wg1_ref[...],
                    preferred_element_type=jnp.float32) + bg1_ref[...], 0.0)
        g = jnp.dot(gh, wg2_ref[...],
                    preferred_element_type=jnp.float32) + bg2_ref[...]
        combined = jnp.concatenate([pooled, g], axis=1)
        hc = jnp.maximum(
            jnp.dot(combined, wc1_ref[...],
                    preferred_element_type=jnp.float32) + bc1_ref[...], 0.0)
        out_ref[...] = jnp.dot(hc, wc2_ref[...],
                               preferred_element_type=jnp.float32) + bc2_ref[...]


def _full(shape):
    return pl.BlockSpec(shape, lambda i: tuple(0 for _ in shape))


def _tc_layer1(x, W1, deg_parts):
    return pl.pallas_call(
        _t1_body,
        grid=(GRID,),
        in_specs=[
            pl.BlockSpec((BLK, D_FEAT), lambda i: (i, 0)),
            _full((D_FEAT, HIDDEN)),
            pl.BlockSpec((NW, BLK), lambda i: (0, i)),
        ],
        out_specs=pl.BlockSpec((BLK, HIDDEN), lambda i: (i, 0)),
        out_shape=jax.ShapeDtypeStruct((N_NODES, HIDDEN), EDGE_DT),
    )(x, W1, deg_parts)


def _tc_layer2(z1, y1, deg_parts, W2, b1):
    return pl.pallas_call(
        _t2_body,
        grid=(GRID,),
        in_specs=[
            pl.BlockSpec((NC, BLK, HIDDEN), lambda i: (0, i, 0)),
            pl.BlockSpec((BLK, HIDDEN), lambda i: (i, 0)),
            pl.BlockSpec((NW, BLK), lambda i: (0, i)),
            _full((HIDDEN, HIDDEN)),
            _full((1, HIDDEN)),
        ],
        out_specs=pl.BlockSpec((BLK, HIDDEN), lambda i: (i, 0)),
        out_shape=jax.ShapeDtypeStruct((N_NODES, HIDDEN), EDGE_DT),
    )(z1, y1, deg_parts, W2, b1)


def _tc_final(z2, y2, deg_parts, batch2d, gf, b2,
              Wg1, bg1, Wg2, bg2, Wc1, bc1, Wc2, bc2):
    return pl.pallas_call(
        _t3_body,
        grid=(GRID,),
        in_specs=[
            pl.BlockSpec((NC, BLK, HIDDEN), lambda i: (0, i, 0)),
            pl.BlockSpec((BLK, HIDDEN), lambda i: (i, 0)),
            pl.BlockSpec((NW, BLK), lambda i: (0, i)),
            pl.BlockSpec((1, BLK), lambda i: (0, i)),
            _full((NUM_GRAPHS, D_GLOBAL)),
            _full((1, HIDDEN)),
            _full((D_GLOBAL, HIDDEN)),
            _full((1, HIDDEN)),
            _full((HIDDEN, HIDDEN)),
            _full((1, HIDDEN)),
            _full((2 * HIDDEN, HIDDEN)),
            _full((1, HIDDEN)),
            _full((HIDDEN, 1)),
            _full((1, 1)),
        ],
        out_specs=_full((NUM_GRAPHS, 1)),
        out_shape=jax.ShapeDtypeStruct((NUM_GRAPHS, 1), jnp.float32),
        scratch_shapes=[
            pltpu.VMEM((NUM_GRAPHS, HIDDEN), jnp.float32),
            pltpu.VMEM((NUM_GRAPHS, 1), jnp.float32),
        ],
    )(z2, y2, deg_parts, batch2d, gf, b2,
      Wg1, bg1, Wg2, bg2, Wc1, bc1, Wc2, bc2)


def kernel(x, edge_index, global_features, batch,
           W1, b1, W2, b2, Wg1, bg1, Wg2, bg2, Wc1, bc1, Wc2, bc2):
    ei = edge_index.astype(jnp.int32)
    src = ei[0]
    dst = ei[1]
    src3 = jnp.concatenate(
        [src.reshape(NW, EPW), jnp.zeros((NW, EPW_PAD), jnp.int32)],
        axis=1).reshape(NW, NCHUNKS, CHUNK)
    pad_rows = (N_NODES + jnp.arange(NW, dtype=jnp.int32) // NC)[:, None]
    dst3 = jnp.concatenate(
        [dst.reshape(NW, EPW),
         jnp.broadcast_to(pad_rows, (NW, EPW_PAD))],
        axis=1).reshape(NW, NCHUNKS, CHUNK)
    batch2d = batch.astype(jnp.int32).reshape(1, N_NODES)
    zeros = jnp.zeros((N_NODES, HIDDEN), EDGE_DT)
    b1r = b1.reshape(1, HIDDEN)
    b2r = b2.reshape(1, HIDDEN)
    bg1r = bg1.reshape(1, HIDDEN)
    bg2r = bg2.reshape(1, HIDDEN)
    bc1r = bc1.reshape(1, HIDDEN)
    bc2r = bc2.reshape(1, 1)

    sc_degree, sc_edge_agg = _sc_kernels()
    deg_parts = sc_degree(dst).reshape(NW, N_NODES)
    y1 = _tc_layer1(x, W1, deg_parts)
    z1 = sc_edge_agg(src3, dst3, y1, zeros)
    y2 = _tc_layer2(z1, y1, deg_parts, W2, b1r)
    z2 = sc_edge_agg(src3, dst3, y2, zeros)
    return _tc_final(z2, y2, deg_parts, batch2d, global_features, b2r,
                     Wg1, bg1r, Wg2, bg2r, Wc1, bc1r, Wc2, bc2r)
